# baseline (speedup 1.0000x reference)
import jax
import jax.numpy as jnp
from jax import lax
from jax.experimental import pallas as pl
from jax.experimental.pallas import tpu as pltpu, tpu_sc as plsc

SEQ_LEN = 200
VOCAB = 100000
EMBED_DIM = 64
BATCH = 4096

NUM_CORES = 2
NUM_SUBCORES = 16
NUM_WORKERS = NUM_CORES * NUM_SUBCORES
ROWS_PER_WORKER = BATCH // NUM_WORKERS
IDX_PER_WORKER = ROWS_PER_WORKER * SEQ_LEN
LANES = 16
CHUNKS_PER_ROW = EMBED_DIM // LANES
SCALE = 8.0
NBUF = 3

GATHER_SPLITS = ((0, 128), (128, 72))


def _sc_kernel(idx_hbm, table_hbm, pos_hbm, out_hbm,
               idx_v, pos_v, hidx, gbuf, gsems, wsems):
    wid = lax.axis_index("s") * NUM_CORES + lax.axis_index("c")
    row_base = wid * ROWS_PER_WORKER
    idx_base = wid * IDX_PER_WORKER

    pltpu.sync_copy(idx_hbm.at[pl.ds(idx_base, IDX_PER_WORKER)],
                    idx_v.at[pl.ds(0, IDX_PER_WORKER)])
    pltpu.sync_copy(pos_hbm, pos_v)

    def prep_hidx(t, k):
        for off in list(range(0, 192, 16)) + [184]:
            v = idx_v[pl.ds(t * SEQ_LEN + off, LANES)]
            hidx[k, pl.ds(off, LANES)] = lax.shift_right_logical(v, 1)

    def issue_gather(t, k):
        for off, n in GATHER_SPLITS:
            pltpu.async_copy(
                table_hbm.at[hidx.at[k, pl.ds(off, n)]],
                gbuf.at[k, pl.ds(off, n)],
                gsems[k],
            )

    def wait_gather(t, k):
        for off, n in GATHER_SPLITS:
            pltpu.make_async_copy(
                table_hbm.at[hidx.at[k, pl.ds(off, n)]],
                gbuf.at[k, pl.ds(off, n)],
                gsems[k],
            ).wait()

    def issue_write(t, k):
        pltpu.async_copy(gbuf.at[k, :, pl.ds(0, EMBED_DIM)],
                         out_hbm.at[row_base + t, :, pl.ds(0, EMBED_DIM)],
                         wsems[k])

    def wait_write(t, k):
        pltpu.make_async_copy(
            gbuf.at[k, :, pl.ds(0, EMBED_DIM)],
            out_hbm.at[row_base + t, :, pl.ds(0, EMBED_DIM)], wsems[k],
        ).wait()

    for k in range(NBUF - 1):
        prep_hidx(k, k)
        issue_gather(k, k)

    def ring_body(g, _):
        u_outer = g * NBUF
        for k in range(NBUF):
            u = u_outer + k
            wait_gather(u, k)

            def per_seq(i, _):
                v = idx_v[pl.ds(u * SEQ_LEN + i, LANES)][0]
                half = (v & 1) * EMBED_DIM
                for j in range(CHUNKS_PER_ROW):
                    x = gbuf[k, i, pl.ds(half + j * LANES, LANES)]
                    gbuf[k, i, pl.ds(j * LANES, LANES)] = (
                        x * SCALE + pos_v[pl.ds(i * EMBED_DIM + j * LANES, LANES)])
                return ()

            lax.fori_loop(0, SEQ_LEN, per_seq, (), unroll=2)

            issue_write(u, k)

            r = u + NBUF - 1
            kr = (k - 1) % NBUF

            @pl.when(r < ROWS_PER_WORKER)
            def _():
                @pl.when(u >= 1)
                def _():
                    wait_write(u - 1, kr)

                prep_hidx(r, kr)
                issue_gather(r, kr)

        return ()

    lax.fori_loop(0, ROWS_PER_WORKER // NBUF, ring_body, ())

    done = ROWS_PER_WORKER // NBUF * NBUF
    for u in range(done, ROWS_PER_WORKER):
        k = u % NBUF
        wait_gather(u, k)

        def per_seq_t(i, _, u=u, k=k):
            v = idx_v[pl.ds(u * SEQ_LEN + i, LANES)][0]
            half = (v & 1) * EMBED_DIM
            for j in range(CHUNKS_PER_ROW):
                x = gbuf[k, i, pl.ds(half + j * LANES, LANES)]
                gbuf[k, i, pl.ds(j * LANES, LANES)] = (
                    x * SCALE + pos_v[pl.ds(i * EMBED_DIM + j * LANES, LANES)])
            return ()

        lax.fori_loop(0, SEQ_LEN, per_seq_t, (), unroll=2)
        issue_write(u, k)
        r = u + NBUF - 1
        if r < ROWS_PER_WORKER:
            wait_write(u - 1, (k - 1) % NBUF)
            prep_hidx(r, (k - 1) % NBUF)
            issue_gather(r, (k - 1) % NBUF)

    for u in range(ROWS_PER_WORKER - NBUF, ROWS_PER_WORKER):
        wait_write(u, u % NBUF)


@jax.jit
def kernel(inputs, token_table, pos_table):
    mesh = plsc.VectorSubcoreMesh(core_axis_name="c", subcore_axis_name="s")
    f = pl.kernel(
        _sc_kernel,
        out_type=jax.ShapeDtypeStruct((BATCH, SEQ_LEN, 2 * EMBED_DIM), jnp.float32),
        mesh=mesh,
        scratch_types=[
            pltpu.VMEM((IDX_PER_WORKER + LANES,), jnp.int32),
            pltpu.VMEM((SEQ_LEN * EMBED_DIM,), jnp.float32),
            pltpu.VMEM((NBUF, SEQ_LEN), jnp.int32),
            pltpu.VMEM((NBUF, SEQ_LEN, 2 * EMBED_DIM), jnp.float32),
            [pltpu.SemaphoreType.DMA] * NBUF,
            [pltpu.SemaphoreType.DMA] * NBUF,
        ],
        compiler_params=pltpu.CompilerParams(use_tc_tiling_on_sc=False),
    )
    return f(inputs.reshape(-1), token_table.reshape(VOCAB // 2, 2 * EMBED_DIM),
             pos_table.reshape(-1))


# speedup vs baseline: 1.0813x; 1.0813x over previous
import jax
import jax.numpy as jnp
from jax import lax
from jax.experimental import pallas as pl
from jax.experimental.pallas import tpu as pltpu, tpu_sc as plsc

SEQ_LEN = 200
VOCAB = 100000
EMBED_DIM = 64
BATCH = 4096

NUM_CORES = 2
NUM_SUBCORES = 16
NUM_WORKERS = NUM_CORES * NUM_SUBCORES
ROWS_PER_WORKER = BATCH // NUM_WORKERS
IDX_PER_WORKER = ROWS_PER_WORKER * SEQ_LEN
LANES = 16
CHUNKS_PER_ROW = EMBED_DIM // LANES
SCALE = 8.0
NBUF = 4

GATHER_SPLITS = ((0, 128), (128, 72))


def _sc_kernel(idx_hbm, table_hbm, out_hbm, idx_v, rows_v, gsems, wsems):
    wid = lax.axis_index("s") * NUM_CORES + lax.axis_index("c")
    row_base = wid * ROWS_PER_WORKER
    idx_base = wid * IDX_PER_WORKER

    pltpu.sync_copy(idx_hbm.at[pl.ds(idx_base, IDX_PER_WORKER)], idx_v)

    def issue_gather(t, k):
        for off, n in GATHER_SPLITS:
            pltpu.async_copy(
                table_hbm.at[idx_v.at[pl.ds(t * SEQ_LEN + off, n)]],
                rows_v.at[k, pl.ds(off, n)],
                gsems[k],
            )

    def wait_gather(t, k):
        for off, n in GATHER_SPLITS:
            pltpu.make_async_copy(
                table_hbm.at[idx_v.at[pl.ds(t * SEQ_LEN + off, n)]],
                rows_v.at[k, pl.ds(off, n)],
                gsems[k],
            ).wait()

    def issue_write(t, k):
        pltpu.async_copy(rows_v.at[k],
                         out_hbm.at[pl.ds((row_base + t) * SEQ_LEN, SEQ_LEN)],
                         wsems[k])

    def wait_write(t, k):
        pltpu.make_async_copy(
            rows_v.at[k],
            out_hbm.at[pl.ds((row_base + t) * SEQ_LEN, SEQ_LEN)],
            wsems[k],
        ).wait()

    for k in range(NBUF - 1):
        issue_gather(k, k)

    def ring_body(g, _):
        u_outer = g * NBUF
        for k in range(NBUF):
            u = u_outer + k
            wait_gather(u, k)
            issue_write(u, k)

            r = u + NBUF - 1
            kr = (k - 1) % NBUF

            @pl.when(r < ROWS_PER_WORKER)
            def _():
                @pl.when(u >= 1)
                def _():
                    wait_write(u - 1, kr)

                issue_gather(r, kr)

        return ()

    lax.fori_loop(0, ROWS_PER_WORKER // NBUF, ring_body, ())

    for k in range(NBUF):
        wait_write(ROWS_PER_WORKER - NBUF + k, k)


TC_BLOCK = 32


def _tc_epilogue(g_ref, pos_ref, out_ref):
    x = g_ref[...].reshape(TC_BLOCK, SEQ_LEN, EMBED_DIM)
    out_ref[...] = x * SCALE + pos_ref[...][None, :, :]


@jax.jit
def kernel(inputs, token_table, pos_table):
    mesh = plsc.VectorSubcoreMesh(core_axis_name="c", subcore_axis_name="s")
    f = pl.kernel(
        _sc_kernel,
        out_type=jax.ShapeDtypeStruct((BATCH * SEQ_LEN, EMBED_DIM),
                                      jnp.float32),
        mesh=mesh,
        scratch_types=[
            pltpu.VMEM((IDX_PER_WORKER,), jnp.int32),
            pltpu.VMEM((NBUF, SEQ_LEN, EMBED_DIM), jnp.float32),
            [pltpu.SemaphoreType.DMA] * NBUF,
            [pltpu.SemaphoreType.DMA] * NBUF,
        ],
        compiler_params=pltpu.CompilerParams(use_tc_tiling_on_sc=False),
    )
    out2d = f(inputs.reshape(-1), token_table)
    out3d = pl.pallas_call(
        _tc_epilogue,
        grid=(BATCH // TC_BLOCK,),
        in_specs=[
            pl.BlockSpec((TC_BLOCK * SEQ_LEN, EMBED_DIM), lambda i: (i, 0)),
            pl.BlockSpec((SEQ_LEN, EMBED_DIM), lambda i: (0, 0)),
        ],
        out_specs=pl.BlockSpec((TC_BLOCK, SEQ_LEN, EMBED_DIM),
                               lambda i: (i, 0, 0)),
        out_shape=jax.ShapeDtypeStruct((BATCH, SEQ_LEN, EMBED_DIM),
                                       jnp.float32),
    )(out2d, pos_table)
    return out3d


# speedup vs baseline: 2.6791x; 2.4776x over previous
import jax
import jax.numpy as jnp
from jax import lax
from jax.experimental import pallas as pl
from jax.experimental.pallas import tpu as pltpu, tpu_sc as plsc

SEQ_LEN = 200
VOCAB = 100000
EMBED_DIM = 64
BATCH = 4096

NUM_CORES = 2
NUM_SUBCORES = 16
NUM_WORKERS = NUM_CORES * NUM_SUBCORES
ROWS_PER_WORKER = BATCH // NUM_WORKERS
IDX_PER_WORKER = ROWS_PER_WORKER * SEQ_LEN
LANES = 16
CHUNKS_PER_ROW = EMBED_DIM // LANES
SCALE = 8.0
NBUF = 4

GATHER_SPLITS = ((0, 128), (128, 72))


def _sc_kernel(idx_hbm, table_hbm, out_hbm, idx_v, rows_v, gsems, wsems):
    wid = lax.axis_index("s") * NUM_CORES + lax.axis_index("c")
    row_base = wid * ROWS_PER_WORKER
    idx_base = wid * IDX_PER_WORKER

    pltpu.sync_copy(idx_hbm.at[pl.ds(idx_base, IDX_PER_WORKER)], idx_v)

    def issue_gather(t, k):
        for off, n in GATHER_SPLITS:
            pltpu.async_copy(
                table_hbm.at[idx_v.at[pl.ds(t * SEQ_LEN + off, n)]],
                rows_v.at[k, pl.ds(off, n)],
                gsems[k],
            )

    def wait_gather(t, k):
        for off, n in GATHER_SPLITS:
            pltpu.make_async_copy(
                table_hbm.at[idx_v.at[pl.ds(t * SEQ_LEN + off, n)]],
                rows_v.at[k, pl.ds(off, n)],
                gsems[k],
            ).wait()

    def issue_write(t, k):
        pltpu.async_copy(rows_v.at[k],
                         out_hbm.at[pl.ds((row_base + t) * SEQ_LEN, SEQ_LEN)],
                         wsems[k])

    def wait_write(t, k):
        pltpu.make_async_copy(
            rows_v.at[k],
            out_hbm.at[pl.ds((row_base + t) * SEQ_LEN, SEQ_LEN)],
            wsems[k],
        ).wait()

    for k in range(NBUF - 1):
        issue_gather(k, k)

    def ring_body(g, _):
        u_outer = g * NBUF
        for k in range(NBUF):
            u = u_outer + k
            wait_gather(u, k)
            issue_write(u, k)

            r = u + NBUF - 1
            kr = (k - 1) % NBUF

            @pl.when(r < ROWS_PER_WORKER)
            def _():
                @pl.when(u >= 1)
                def _():
                    wait_write(u - 1, kr)

                issue_gather(r, kr)

        return ()

    lax.fori_loop(0, ROWS_PER_WORKER // NBUF, ring_body, ())

    for k in range(NBUF):
        wait_write(ROWS_PER_WORKER - NBUF + k, k)


TC_BLOCK = 64


def _tc_epilogue(g_ref, pos_ref, out_ref):
    x = g_ref[...].reshape(2 * TC_BLOCK, SEQ_LEN // 2, 2 * EMBED_DIM)
    y = jnp.transpose(x, (1, 2, 0))
    out_ref[...] = y * SCALE + pos_ref[...][:, :, None]


@jax.jit
def kernel(inputs, token_table, pos_table):
    mesh = plsc.VectorSubcoreMesh(core_axis_name="c", subcore_axis_name="s")
    f = pl.kernel(
        _sc_kernel,
        out_type=jax.ShapeDtypeStruct((BATCH * SEQ_LEN, EMBED_DIM),
                                      jnp.float32),
        mesh=mesh,
        scratch_types=[
            pltpu.VMEM((IDX_PER_WORKER,), jnp.int32),
            pltpu.VMEM((NBUF, SEQ_LEN, EMBED_DIM), jnp.float32),
            [pltpu.SemaphoreType.DMA] * NBUF,
            [pltpu.SemaphoreType.DMA] * NBUF,
        ],
        compiler_params=pltpu.CompilerParams(use_tc_tiling_on_sc=False),
    )
    out2d = f(inputs.reshape(-1), token_table)
    g3 = out2d.reshape(BATCH // 2, SEQ_LEN, 2 * EMBED_DIM)
    pos2 = pos_table.reshape(SEQ_LEN // 2, 2 * EMBED_DIM)
    out_t = pl.pallas_call(
        _tc_epilogue,
        grid=(BATCH // 2 // TC_BLOCK,),
        in_specs=[
            pl.BlockSpec((TC_BLOCK, SEQ_LEN, 2 * EMBED_DIM),
                         lambda i: (i, 0, 0)),
            pl.BlockSpec((SEQ_LEN // 2, 2 * EMBED_DIM), lambda i: (0, 0)),
        ],
        out_specs=pl.BlockSpec((SEQ_LEN // 2, 2 * EMBED_DIM, 2 * TC_BLOCK),
                               lambda i: (0, 0, i)),
        out_shape=jax.ShapeDtypeStruct((SEQ_LEN // 2, 2 * EMBED_DIM, BATCH),
                                       jnp.float32),
        compiler_params=pltpu.CompilerParams(
            vmem_limit_bytes=100 * 1024 * 1024),
    )(g3, pos2)
    return out_t.reshape(SEQ_LEN, EMBED_DIM, BATCH).transpose(2, 0, 1)
